# baseline (speedup 1.0000x reference)
import functools

import jax
import jax.numpy as jnp
from jax import lax
from jax.experimental import pallas as pl
from jax.experimental.pallas import tpu as pltpu
from jax.experimental.pallas import tpu_sc as plsc

_SENT = 1.0e4
_INIT = 1.0e9


def _sc_sqrt(v):
    iv = plsc.bitcast(v, jnp.int32)
    r = plsc.bitcast(0x5F3759DF - (iv >> 1), jnp.float32)
    for _ in range(3):
        r = r * (1.5 - 0.5 * v * r * r)
    return v * r


def _sc_body(pred_hbm, lab_hbm, out_hbm, predv, labv, ffv, wfv, bnv, sbv,
             gtv, outv):
    io = lax.iota(jnp.int32, 16)
    m = lax.axis_index("s")
    p = m % 8
    dirn = m // 8
    jidx = p // 4
    i_img = p % 4

    pltpu.sync_copy(pred_hbm.at[pl.ds(i_img * 12288, 12288)], predv)
    pltpu.sync_copy(
        lab_hbm.at[pl.ds((i_img * 3 + jidx + 1) * 4096, 4096)], labv)

    jv = jnp.full((16,), jidx, jnp.int32)
    dv = jnp.full((16,), dirn, jnp.int32)

    def p1_body(y, carry):
        ca, cb, nw = carry
        for xg in range(4):
            off = y * 64 + xg * 16
            p0 = predv[pl.ds(off, 16)]
            p1 = predv[pl.ds(4096 + off, 16)]
            p2 = predv[pl.ds(8192 + off, 16)]
            b = labv[pl.ds(off, 16)].astype(jnp.float32)
            oh1 = jnp.where((p1 > p0) & (p1 >= p2), 1.0, 0.0)
            oh2 = jnp.where((p2 > p0) & (p2 > p1), 1.0, 0.0)
            a = jnp.where(jv == 0, oh1, oh2)
            f_fld = jnp.where(dv == 0, b, a)
            g_fld = jnp.where(dv == 0, a, b)
            w = g_fld * (1.0 - f_fld)
            ffv[pl.ds(off, 16)] = f_fld
            wfv[pl.ds(off, 16)] = w
            ca = ca + a
            cb = cb + b
            nw = nw + w
        return ca, cb, nw

    z16 = jnp.zeros((16,), jnp.float32)
    ca_v, cb_v, nw_v = lax.fori_loop(0, 64, p1_body, (z16, z16, z16))

    def p2_body(y, ne):
        ym = jnp.maximum(y - 1, 0)
        yp = jnp.minimum(y + 1, 63)
        um = jnp.where(jnp.full((16,), y, jnp.int32) > 0, 1.0, 0.0)
        dm = jnp.where(jnp.full((16,), y, jnp.int32) < 63, 1.0, 0.0)
        for xg in range(4):
            gx = io + xg * 16
            lm = jnp.where(gx > 0, 1.0, 0.0)
            rm = jnp.where(gx < 63, 1.0, 0.0)
            off = y * 64 + xg * 16
            c = ffv[pl.ds(off, 16)]
            up = ffv[pl.ds(ym * 64 + xg * 16, 16)] * um
            dn = ffv[pl.ds(yp * 64 + xg * 16, 16)] * dm
            xi = off + io
            li = jnp.maximum(xi - 1, y * 64)
            ri = jnp.minimum(xi + 1, y * 64 + 63)
            lf = plsc.load_gather(ffv, [li]) * lm
            rf = plsc.load_gather(ffv, [ri]) * rm
            nb = c + up + dn + lf + rf
            bv = jnp.where(c * (5.0 - nb) > 0.0, 1.0, 0.0)
            bnv[pl.ds(off, 16)] = bv
            ne = ne + bv
        return ne

    ne_v = lax.fori_loop(0, 64, p2_body, z16)

    def p3_body(y, f):
        out = []
        for xg in range(4):
            off = y * 64 + xg * 16
            pen = bnv[pl.ds(off, 16)]
            fn = (f[xg] + 1.0) * (1.0 - pen)
            sbv[pl.ds(off, 16)] = fn
            out.append(fn)
        return tuple(out)

    s16 = jnp.full((16,), _SENT, jnp.float32)
    lax.fori_loop(0, 64, p3_body, (s16, s16, s16, s16))

    def p4_body(t, bw):
        y = 63 - t
        out = []
        for xg in range(4):
            off = y * 64 + xg * 16
            pen = bnv[pl.ds(off, 16)]
            bn = (bw[xg] + 1.0) * (1.0 - pen)
            near = jnp.minimum(bn, sbv[pl.ds(off, 16)])
            g = near * near
            idx = io * 64 + (xg * 1024 + y)
            plsc.store_scatter(gtv, [idx], g)
            out.append(bn)
        return tuple(out)

    lax.fori_loop(0, 64, p4_body, (s16, s16, s16, s16))

    def chunk_body(cidx, acc):
        xb = cidx * 8
        init = tuple(jnp.full((16,), _INIT, jnp.float32) for _ in range(32))

        def inner(xp, st):
            rows = [gtv[pl.ds(xp * 64 + q * 16, 16)] for q in range(4)]
            base = (xb - xp).astype(jnp.float32)
            new = []
            for k in range(8):
                dk = base + float(k)
                add = jnp.full((16,), dk * dk)
                for q in range(4):
                    new.append(jnp.minimum(st[k * 4 + q], rows[q] + add))
            return tuple(new)

        st = lax.fori_loop(0, 64, inner, init)
        for k in range(8):
            x = xb + k
            for q in range(4):
                s = _sc_sqrt(st[k * 4 + q])
                widx = io * 64 + (q * 1024 + x)
                wv = plsc.load_gather(wfv, [widx])
                acc = acc + s * wv
        return acc

    acc_v = lax.fori_loop(0, 8, chunk_body, z16)

    s_sum = jnp.sum(acc_v)
    res = jnp.where(io == 0, s_sum, 0.0)
    res = res + jnp.where(io == 1, jnp.sum(ne_v), 0.0)
    res = res + jnp.where(io == 2, jnp.sum(nw_v), 0.0)
    res = res + jnp.where(io == 3, jnp.sum(ca_v), 0.0)
    res = res + jnp.where(io == 4, jnp.sum(cb_v), 0.0)
    outv[...] = res
    pltpu.sync_copy(outv, out_hbm.at[pl.ds(m * 16, 16)])


_sc_call = pl.kernel(
    _sc_body,
    out_type=jax.ShapeDtypeStruct((256,), jnp.float32),
    mesh=plsc.VectorSubcoreMesh(core_axis_name="c", subcore_axis_name="s",
                                num_cores=1, num_subcores=16),
    compiler_params=pltpu.CompilerParams(needs_layout_passes=False),
    scratch_types=[
        pltpu.VMEM((12288,), jnp.float32),
        pltpu.VMEM((4096,), jnp.int32),
        pltpu.VMEM((4096,), jnp.float32),
        pltpu.VMEM((4096,), jnp.float32),
        pltpu.VMEM((4096,), jnp.float32),
        pltpu.VMEM((4096,), jnp.float32),
        pltpu.VMEM((4096,), jnp.float32),
        pltpu.VMEM((16,), jnp.float32),
    ],
)


def _asm_body(p_ref, hd_ref, fail_ref):
    P = p_ref[...]
    hd = [None] * 8
    fail = [None] * 8
    for p in range(8):
        s_f = P[p, 0]
        s_b = P[8 + p, 0]
        ne_f = P[p, 1]
        nw_f = P[p, 2]
        ne_b = P[8 + p, 1]
        nw_b = P[8 + p, 2]
        ca = P[p, 3]
        cb = P[p, 4]
        hd_f = jnp.where((nw_f > 0) & (ne_f > 0),
                         s_f / jnp.maximum(ca, 1.0), 0.0)
        hd_b = jnp.where((nw_b > 0) & (ne_b > 0),
                         s_b / jnp.maximum(cb, 1.0), 0.0)
        hh = jnp.maximum(hd_f, hd_b)
        hd[p] = jnp.where(ca > 0, hh, 32.0)
        fail[p] = jnp.where(ca > 0, 0.0, 1.0)

    f1 = fail[0] + fail[1] + fail[2] + fail[3]
    f2 = fail[4] + fail[5] + fail[6] + fail[7]

    rr = lax.broadcasted_iota(jnp.int32, (8, 128), 0)
    cc = lax.broadcasted_iota(jnp.int32, (8, 128), 1)
    hdpad = jnp.zeros((8, 128), jnp.float32)
    for i in range(4):
        h1 = hd[i]
        h2 = hd[4 + i]
        for col, val in [(1, h1), (2, h2), (3, (h1 + h2) / 3.0),
                         (4, h1 / 2.0)]:
            hdpad = hdpad + jnp.where((rr == i) & (cc == col), val, 0.0)
    hd_ref[...] = hdpad

    fpad = jnp.zeros((8, 128), jnp.float32)
    for col, val in [(1, f1), (2, f2), (3, (f1 + f2) / 3.0),
                     (4, (f1 + f2) / 2.0)]:
        fpad = fpad + jnp.where((rr == 0) & (cc == col), val, 0.0)
    fail_ref[...] = fpad


def kernel(predictions, labels):
    partials = _sc_call(predictions.reshape(-1), labels.reshape(-1))
    hdpad, fpad = pl.pallas_call(
        _asm_body,
        out_shape=[
            jax.ShapeDtypeStruct((8, 128), jnp.float32),
            jax.ShapeDtypeStruct((8, 128), jnp.float32),
        ],
    )(partials.reshape(16, 16))
    return hdpad[:4, :5], fpad[0, :5]


# speedup vs baseline: 1.1170x; 1.1170x over previous
import jax
import jax.numpy as jnp
from jax import lax
from jax.experimental import pallas as pl
from jax.experimental.pallas import tpu as pltpu
from jax.experimental.pallas import tpu_sc as plsc

_BIG = 1e9
_SENT = 1.0e4
_INIT = 1.0e9


def _sc_sqrt(v):
    iv = plsc.bitcast(v, jnp.int32)
    r = plsc.bitcast(0x5F3759DF - (iv >> 1), jnp.float32)
    for _ in range(3):
        r = r * (1.5 - 0.5 * v * r * r)
    return v * r


def _sc_body(pred_hbm, lab_hbm, out_hbm, predv, labv, ffv, wfv, bnv, sbv,
             gtv, outv):
    io = lax.iota(jnp.int32, 16)
    m = lax.axis_index("s")
    p = m % 8
    h = m // 8
    jidx = p // 4
    i_img = p % 4

    pltpu.sync_copy(pred_hbm.at[pl.ds(i_img * 12288, 12288)], predv)
    pltpu.sync_copy(
        lab_hbm.at[pl.ds((i_img * 3 + jidx + 1) * 4096, 4096)], labv)

    jv = jnp.full((16,), jidx, jnp.int32)

    def p1_body(y, carry):
        ca, cb, nw = carry
        for xg in range(4):
            off = y * 64 + xg * 16
            p0 = predv[pl.ds(off, 16)]
            p1 = predv[pl.ds(4096 + off, 16)]
            p2 = predv[pl.ds(8192 + off, 16)]
            b = labv[pl.ds(off, 16)].astype(jnp.float32)
            oh1 = jnp.where((p1 > p0) & (p1 >= p2), 1.0, 0.0)
            oh2 = jnp.where((p2 > p0) & (p2 > p1), 1.0, 0.0)
            a = jnp.where(jv == 0, oh1, oh2)
            w = b * (1.0 - a)
            ffv[pl.ds(off, 16)] = a
            wfv[pl.ds(off, 16)] = w
            ca = ca + a
            cb = cb + b
            nw = nw + w
        return ca, cb, nw

    z16 = jnp.zeros((16,), jnp.float32)
    ca_v, cb_v, nw_v = lax.fori_loop(0, 64, p1_body, (z16, z16, z16))

    def p2_body(y, ne):
        ym = jnp.maximum(y - 1, 0)
        yp = jnp.minimum(y + 1, 63)
        um = jnp.where(jnp.full((16,), y, jnp.int32) > 0, 1.0, 0.0)
        dm = jnp.where(jnp.full((16,), y, jnp.int32) < 63, 1.0, 0.0)
        for xg in range(4):
            gx = io + xg * 16
            lm = jnp.where(gx > 0, 1.0, 0.0)
            rm = jnp.where(gx < 63, 1.0, 0.0)
            off = y * 64 + xg * 16
            c = ffv[pl.ds(off, 16)]
            up = ffv[pl.ds(ym * 64 + xg * 16, 16)] * um
            dn = ffv[pl.ds(yp * 64 + xg * 16, 16)] * dm
            xi = off + io
            li = jnp.maximum(xi - 1, y * 64)
            ri = jnp.minimum(xi + 1, y * 64 + 63)
            lf = plsc.load_gather(ffv, [li]) * lm
            rf = plsc.load_gather(ffv, [ri]) * rm
            nb = c + up + dn + lf + rf
            bv = jnp.where(c * (5.0 - nb) > 0.0, 1.0, 0.0)
            bnv[pl.ds(off, 16)] = bv
            ne = ne + bv
        return ne

    ne_v = lax.fori_loop(0, 64, p2_body, z16)

    def p3_body(y, f):
        out = []
        for xg in range(4):
            off = y * 64 + xg * 16
            pen = bnv[pl.ds(off, 16)]
            fn = (f[xg] + 1.0) * (1.0 - pen)
            sbv[pl.ds(off, 16)] = fn
            out.append(fn)
        return tuple(out)

    s16 = jnp.full((16,), _SENT, jnp.float32)
    lax.fori_loop(0, 64, p3_body, (s16, s16, s16, s16))

    def p4_body(t, bw):
        y = 63 - t
        out = []
        for xg in range(4):
            off = y * 64 + xg * 16
            pen = bnv[pl.ds(off, 16)]
            bn = (bw[xg] + 1.0) * (1.0 - pen)
            near = jnp.minimum(bn, sbv[pl.ds(off, 16)])
            g = near * near
            idx = io * 64 + (xg * 1024 + y)
            plsc.store_scatter(gtv, [idx], g)
            out.append(bn)
        return tuple(out)

    lax.fori_loop(0, 64, p4_body, (s16, s16, s16, s16))

    x0 = h * 32

    def chunk_body(cidx, acc):
        xb = x0 + cidx * 8
        init = tuple(jnp.full((16,), _INIT, jnp.float32) for _ in range(32))

        def inner(xp, st):
            rows = [gtv[pl.ds(xp * 64 + q * 16, 16)] for q in range(4)]
            base = (xb - xp).astype(jnp.float32)
            new = []
            for k in range(8):
                dk = base + float(k)
                add = jnp.full((16,), dk * dk)
                for q in range(4):
                    new.append(jnp.minimum(st[k * 4 + q], rows[q] + add))
            return tuple(new)

        st = lax.fori_loop(0, 64, inner, init)
        for k in range(8):
            x = xb + k
            for q in range(4):
                s = _sc_sqrt(st[k * 4 + q])
                widx = io * 64 + (q * 1024 + x)
                wv = plsc.load_gather(wfv, [widx])
                acc = acc + s * wv
        return acc

    acc_v = lax.fori_loop(0, 4, chunk_body, z16)

    res = jnp.where(io == 0, jnp.sum(acc_v), 0.0)
    res = res + jnp.where(io == 1, jnp.sum(ne_v), 0.0)
    res = res + jnp.where(io == 2, jnp.sum(nw_v), 0.0)
    res = res + jnp.where(io == 3, jnp.sum(ca_v), 0.0)
    res = res + jnp.where(io == 4, jnp.sum(cb_v), 0.0)
    outv[...] = res
    pltpu.sync_copy(outv, out_hbm.at[pl.ds(m * 16, 16)])


_sc_call = pl.kernel(
    _sc_body,
    out_type=jax.ShapeDtypeStruct((256,), jnp.float32),
    mesh=plsc.VectorSubcoreMesh(core_axis_name="c", subcore_axis_name="s",
                                num_cores=1, num_subcores=16),
    compiler_params=pltpu.CompilerParams(needs_layout_passes=False),
    scratch_types=[
        pltpu.VMEM((12288,), jnp.float32),
        pltpu.VMEM((4096,), jnp.int32),
        pltpu.VMEM((4096,), jnp.float32),
        pltpu.VMEM((4096,), jnp.float32),
        pltpu.VMEM((4096,), jnp.float32),
        pltpu.VMEM((4096,), jnp.float32),
        pltpu.VMEM((4096,), jnp.float32),
        pltpu.VMEM((16,), jnp.float32),
    ],
)


def _boundary_mask(mk):
    z_row = jnp.zeros((1, 64), jnp.float32)
    z_col = jnp.zeros((64, 1), jnp.float32)
    new = mk
    new = new + jnp.concatenate([mk[1:, :], z_row], axis=0)
    new = new + jnp.concatenate([z_row, mk[:-1, :]], axis=0)
    new = new + jnp.concatenate([mk[:, 1:], z_col], axis=1)
    new = new + jnp.concatenate([z_col, mk[:, :-1]], axis=1)
    return jnp.where(mk * (5.0 - new) > 0.0, 1.0, 0.0)


def _minplus_pass(pen):
    t_idx = jax.lax.broadcasted_iota(jnp.int32, (64, 1), 0).astype(jnp.float32)
    out = jnp.full(pen.shape, 4.0 * _BIG, jnp.float32)
    for s in range(64):
        d2 = (t_idx - float(s)) ** 2
        out = jnp.minimum(out, pen[s:s + 1, :] + d2)
    return out


def _transpose_blocks(x, eye, nblk):
    outs = []
    for mi in range(nblk):
        blk = x[:, mi * 64:(mi + 1) * 64]
        outs.append(
            jax.lax.dot_general(blk, eye, (((0,), (0,)), ((), ())),
                                preferred_element_type=jnp.float32))
    return jnp.concatenate(outs, axis=1)


def _tc_fwd_body(pred_ref, lab_ref, out_ref):
    pred = pred_ref[...]
    lab = lab_ref[...].astype(jnp.float32)

    p0, p1, p2 = pred[:, 0], pred[:, 1], pred[:, 2]
    a_cls = [
        None,
        jnp.where((p1 > p0) & (p1 >= p2), 1.0, 0.0),
        jnp.where((p2 > p0) & (p2 > p1), 1.0, 0.0),
    ]

    edt_masks = [None] * 8
    w_masks = [None] * 8
    count_a = [None] * 8
    count_b = [None] * 8
    for jidx in range(2):
        for i in range(4):
            p = jidx * 4 + i
            a = a_cls[jidx + 1][i]
            b = lab[i, jidx + 1]
            edt_masks[p] = _boundary_mask(b)
            w_masks[p] = a * (1.0 - b)
            count_a[p] = jnp.sum(a)
            count_b[p] = jnp.sum(b)

    pen = jnp.concatenate([(1.0 - mm) * _BIG for mm in edt_masks], axis=1)
    w2d = jnp.concatenate(w_masks, axis=1)

    r64 = jax.lax.broadcasted_iota(jnp.int32, (64, 64), 0)
    c64 = jax.lax.broadcasted_iota(jnp.int32, (64, 64), 1)
    eye = (r64 == c64).astype(jnp.float32)

    g = _minplus_pass(pen)
    gt = _transpose_blocks(g, eye, 8)
    d2t = _minplus_pass(gt)
    wt = _transpose_blocks(w2d, eye, 8)

    prod = jnp.sqrt(jnp.maximum(d2t, 0.0)) * wt

    rr = lax.broadcasted_iota(jnp.int32, (8, 128), 0)
    cc = lax.broadcasted_iota(jnp.int32, (8, 128), 1)
    out = jnp.zeros((8, 128), jnp.float32)
    for p in range(8):
        blk = slice(p * 64, (p + 1) * 64)
        s_f = jnp.sum(prod[:, blk])
        ne_f = jnp.sum(pen[:, blk] < 1.0)
        nw_f = jnp.sum(w2d[:, blk])
        for col, val in [(0, s_f), (1, ne_f), (2, nw_f), (3, count_a[p]),
                         (4, count_b[p])]:
            out = out + jnp.where((rr == p) & (cc == col), val, 0.0)
    out_ref[...] = out


def _asm_body(t1_ref, scp_ref, hd_ref, fail_ref):
    T1 = t1_ref[...]
    SCP = scp_ref[...]
    hd = [None] * 8
    fail = [None] * 8
    for p in range(8):
        s_f = T1[p, 0]
        ne_f = T1[p, 1]
        nw_f = T1[p, 2]
        ca = T1[p, 3]
        cb = T1[p, 4]
        s_b = SCP[p, 0] + SCP[8 + p, 0]
        ne_b = SCP[p, 1]
        nw_b = SCP[p, 2]
        hd_f = jnp.where((nw_f > 0) & (ne_f > 0),
                         s_f / jnp.maximum(ca, 1.0), 0.0)
        hd_b = jnp.where((nw_b > 0) & (ne_b > 0),
                         s_b / jnp.maximum(cb, 1.0), 0.0)
        hh = jnp.maximum(hd_f, hd_b)
        hd[p] = jnp.where(ca > 0, hh, 32.0)
        fail[p] = jnp.where(ca > 0, 0.0, 1.0)

    f1 = fail[0] + fail[1] + fail[2] + fail[3]
    f2 = fail[4] + fail[5] + fail[6] + fail[7]

    rr = lax.broadcasted_iota(jnp.int32, (8, 128), 0)
    cc = lax.broadcasted_iota(jnp.int32, (8, 128), 1)
    hdpad = jnp.zeros((8, 128), jnp.float32)
    for i in range(4):
        h1 = hd[i]
        h2 = hd[4 + i]
        for col, val in [(1, h1), (2, h2), (3, (h1 + h2) / 3.0),
                         (4, h1 / 2.0)]:
            hdpad = hdpad + jnp.where((rr == i) & (cc == col), val, 0.0)
    hd_ref[...] = hdpad

    fpad = jnp.zeros((8, 128), jnp.float32)
    for col, val in [(1, f1), (2, f2), (3, (f1 + f2) / 3.0),
                     (4, (f1 + f2) / 2.0)]:
        fpad = fpad + jnp.where((rr == 0) & (cc == col), val, 0.0)
    fail_ref[...] = fpad


def kernel(predictions, labels):
    sc_partials = _sc_call(predictions.reshape(-1), labels.reshape(-1))
    t1 = pl.pallas_call(
        _tc_fwd_body,
        out_shape=jax.ShapeDtypeStruct((8, 128), jnp.float32),
    )(predictions, labels)
    hdpad, fpad = pl.pallas_call(
        _asm_body,
        out_shape=[
            jax.ShapeDtypeStruct((8, 128), jnp.float32),
            jax.ShapeDtypeStruct((8, 128), jnp.float32),
        ],
    )(t1, sc_partials.reshape(16, 16))
    return hdpad[:4, :5], fpad[0, :5]


# speedup vs baseline: 1.1837x; 1.0597x over previous
import jax
import jax.numpy as jnp
from jax import lax
from jax.experimental import pallas as pl
from jax.experimental.pallas import tpu as pltpu
from jax.experimental.pallas import tpu_sc as plsc

_BIG = 1e9
_SENT = 1.0e4
_INIT = 1.0e9


def _sc_sqrt(v):
    iv = plsc.bitcast(v, jnp.int32)
    r = plsc.bitcast(0x5F3759DF - (iv >> 1), jnp.float32)
    for _ in range(3):
        r = r * (1.5 - 0.5 * v * r * r)
    return v * r


def _sc_body(pred_hbm, lab_hbm, out_hbm, predv, labv, ffv, wfv, bnv, sbv,
             gtv, outv):
    io = lax.iota(jnp.int32, 16)
    m = lax.axis_index("s")
    i_img = m // 4
    q = m % 4

    pltpu.sync_copy(pred_hbm.at[pl.ds(i_img * 12288, 12288)], predv)
    pltpu.sync_copy(
        lab_hbm.at[pl.ds((i_img * 3 + 1) * 4096, 4096)], labv)

    def p1_body(y, carry):
        ca, cb, nw = carry
        for xg in range(4):
            off = y * 64 + xg * 16
            p0 = predv[pl.ds(off, 16)]
            p1 = predv[pl.ds(4096 + off, 16)]
            p2 = predv[pl.ds(8192 + off, 16)]
            b = labv[pl.ds(off, 16)].astype(jnp.float32)
            a = jnp.where((p1 > p0) & (p1 >= p2), 1.0, 0.0)
            w = b * (1.0 - a)
            ffv[pl.ds(off, 16)] = a
            wfv[pl.ds(off, 16)] = w
            ca = ca + a
            cb = cb + b
            nw = nw + w
        return ca, cb, nw

    z16 = jnp.zeros((16,), jnp.float32)
    ca_v, cb_v, nw_v = lax.fori_loop(0, 64, p1_body, (z16, z16, z16))

    def p2_body(y, ne):
        ym = jnp.maximum(y - 1, 0)
        yp = jnp.minimum(y + 1, 63)
        um = jnp.where(jnp.full((16,), y, jnp.int32) > 0, 1.0, 0.0)
        dm = jnp.where(jnp.full((16,), y, jnp.int32) < 63, 1.0, 0.0)
        for xg in range(4):
            gx = io + xg * 16
            lm = jnp.where(gx > 0, 1.0, 0.0)
            rm = jnp.where(gx < 63, 1.0, 0.0)
            off = y * 64 + xg * 16
            c = ffv[pl.ds(off, 16)]
            up = ffv[pl.ds(ym * 64 + xg * 16, 16)] * um
            dn = ffv[pl.ds(yp * 64 + xg * 16, 16)] * dm
            xi = off + io
            li = jnp.maximum(xi - 1, y * 64)
            ri = jnp.minimum(xi + 1, y * 64 + 63)
            lf = plsc.load_gather(ffv, [li]) * lm
            rf = plsc.load_gather(ffv, [ri]) * rm
            nb = c + up + dn + lf + rf
            bv = jnp.where(c * (5.0 - nb) > 0.0, 1.0, 0.0)
            bnv[pl.ds(off, 16)] = bv
            ne = ne + bv
        return ne

    ne_v = lax.fori_loop(0, 64, p2_body, z16)

    def p3_body(y, f):
        out = []
        for xg in range(4):
            off = y * 64 + xg * 16
            pen = bnv[pl.ds(off, 16)]
            fn = (f[xg] + 1.0) * (1.0 - pen)
            sbv[pl.ds(off, 16)] = fn
            out.append(fn)
        return tuple(out)

    s16 = jnp.full((16,), _SENT, jnp.float32)
    lax.fori_loop(0, 64, p3_body, (s16, s16, s16, s16))

    def p4_body(t, bw):
        y = 63 - t
        out = []
        for xg in range(4):
            off = y * 64 + xg * 16
            pen = bnv[pl.ds(off, 16)]
            bn = (bw[xg] + 1.0) * (1.0 - pen)
            near = jnp.minimum(bn, sbv[pl.ds(off, 16)])
            g = near * near
            idx = io * 64 + (xg * 1024 + y)
            plsc.store_scatter(gtv, [idx], g)
            out.append(bn)
        return tuple(out)

    lax.fori_loop(0, 64, p4_body, (s16, s16, s16, s16))

    x0 = q * 16

    def chunk_body(cidx, acc):
        xb = x0 + cidx * 8
        init = tuple(jnp.full((16,), _INIT, jnp.float32) for _ in range(32))

        def inner(xp, st):
            rows = [gtv[pl.ds(xp * 64 + q * 16, 16)] for q in range(4)]
            base = (xb - xp).astype(jnp.float32)
            new = []
            for k in range(8):
                dk = base + float(k)
                add = jnp.full((16,), dk * dk)
                for q in range(4):
                    new.append(jnp.minimum(st[k * 4 + q], rows[q] + add))
            return tuple(new)

        st = lax.fori_loop(0, 64, inner, init)
        for k in range(8):
            x = xb + k
            for q in range(4):
                s = _sc_sqrt(st[k * 4 + q])
                widx = io * 64 + (q * 1024 + x)
                wv = plsc.load_gather(wfv, [widx])
                acc = acc + s * wv
        return acc

    acc_v = lax.fori_loop(0, 2, chunk_body, z16)

    res = jnp.where(io == 0, jnp.sum(acc_v), 0.0)
    res = res + jnp.where(io == 1, jnp.sum(ne_v), 0.0)
    res = res + jnp.where(io == 2, jnp.sum(nw_v), 0.0)
    res = res + jnp.where(io == 3, jnp.sum(ca_v), 0.0)
    res = res + jnp.where(io == 4, jnp.sum(cb_v), 0.0)
    outv[...] = res
    pltpu.sync_copy(outv, out_hbm.at[pl.ds(m * 16, 16)])


_sc_call = pl.kernel(
    _sc_body,
    out_type=jax.ShapeDtypeStruct((256,), jnp.float32),
    mesh=plsc.VectorSubcoreMesh(core_axis_name="c", subcore_axis_name="s",
                                num_cores=1, num_subcores=16),
    compiler_params=pltpu.CompilerParams(needs_layout_passes=False),
    scratch_types=[
        pltpu.VMEM((12288,), jnp.float32),
        pltpu.VMEM((4096,), jnp.int32),
        pltpu.VMEM((4096,), jnp.float32),
        pltpu.VMEM((4096,), jnp.float32),
        pltpu.VMEM((4096,), jnp.float32),
        pltpu.VMEM((4096,), jnp.float32),
        pltpu.VMEM((4096,), jnp.float32),
        pltpu.VMEM((16,), jnp.float32),
    ],
)


def _boundary_mask(mk):
    z_row = jnp.zeros((1, 64), jnp.float32)
    z_col = jnp.zeros((64, 1), jnp.float32)
    new = mk
    new = new + jnp.concatenate([mk[1:, :], z_row], axis=0)
    new = new + jnp.concatenate([z_row, mk[:-1, :]], axis=0)
    new = new + jnp.concatenate([mk[:, 1:], z_col], axis=1)
    new = new + jnp.concatenate([z_col, mk[:, :-1]], axis=1)
    return jnp.where(mk * (5.0 - new) > 0.0, 1.0, 0.0)


def _minplus_pass(pen):
    t_idx = jax.lax.broadcasted_iota(jnp.int32, (64, 1), 0).astype(jnp.float32)
    out = jnp.full(pen.shape, 4.0 * _BIG, jnp.float32)
    for s in range(64):
        d2 = (t_idx - float(s)) ** 2
        out = jnp.minimum(out, pen[s:s + 1, :] + d2)
    return out


def _transpose_blocks(x, eye, nblk):
    outs = []
    for mi in range(nblk):
        blk = x[:, mi * 64:(mi + 1) * 64]
        outs.append(
            jax.lax.dot_general(blk, eye, (((0,), (0,)), ((), ())),
                                preferred_element_type=jnp.float32))
    return jnp.concatenate(outs, axis=1)


def _tc_fwd_body(pred_ref, lab_ref, out_ref):
    pred = pred_ref[...]
    lab = lab_ref[...].astype(jnp.float32)

    p0, p1, p2 = pred[:, 0], pred[:, 1], pred[:, 2]
    a_cls = [
        None,
        jnp.where((p1 > p0) & (p1 >= p2), 1.0, 0.0),
        jnp.where((p2 > p0) & (p2 > p1), 1.0, 0.0),
    ]

    edt_masks = [None] * 12
    w_masks = [None] * 12
    count_a = [None] * 8
    count_b = [None] * 8
    for jidx in range(2):
        for i in range(4):
            p = jidx * 4 + i
            a = a_cls[jidx + 1][i]
            b = lab[i, jidx + 1]
            edt_masks[p] = _boundary_mask(b)
            w_masks[p] = a * (1.0 - b)
            count_a[p] = jnp.sum(a)
            count_b[p] = jnp.sum(b)
            if jidx == 1:
                edt_masks[8 + i] = _boundary_mask(a)
                w_masks[8 + i] = b * (1.0 - a)

    pen = jnp.concatenate([(1.0 - mm) * _BIG for mm in edt_masks], axis=1)
    w2d = jnp.concatenate(w_masks, axis=1)

    r64 = jax.lax.broadcasted_iota(jnp.int32, (64, 64), 0)
    c64 = jax.lax.broadcasted_iota(jnp.int32, (64, 64), 1)
    eye = (r64 == c64).astype(jnp.float32)

    g = _minplus_pass(pen)
    gt = _transpose_blocks(g, eye, 12)
    d2t = _minplus_pass(gt)
    wt = _transpose_blocks(w2d, eye, 12)

    prod = jnp.sqrt(jnp.maximum(d2t, 0.0)) * wt

    rr = lax.broadcasted_iota(jnp.int32, (16, 128), 0)
    cc = lax.broadcasted_iota(jnp.int32, (16, 128), 1)
    out = jnp.zeros((16, 128), jnp.float32)
    for p in range(12):
        blk = slice(p * 64, (p + 1) * 64)
        s_p = jnp.sum(prod[:, blk])
        ne_p = jnp.sum(pen[:, blk] < 1.0)
        nw_p = jnp.sum(w2d[:, blk])
        vals = [(0, s_p), (1, ne_p), (2, nw_p)]
        if p < 8:
            vals += [(3, count_a[p]), (4, count_b[p])]
        for col, val in vals:
            out = out + jnp.where((rr == p) & (cc == col), val, 0.0)
    out_ref[...] = out


def _asm_body(t1_ref, scp_ref, hd_ref, fail_ref):
    T1 = t1_ref[...]
    SCP = scp_ref[...]
    hd = [None] * 8
    fail = [None] * 8
    for p in range(8):
        s_f = T1[p, 0]
        ne_f = T1[p, 1]
        nw_f = T1[p, 2]
        ca = T1[p, 3]
        cb = T1[p, 4]
        if p < 4:
            s_b = (SCP[4 * p, 0] + SCP[4 * p + 1, 0] + SCP[4 * p + 2, 0]
                   + SCP[4 * p + 3, 0])
            ne_b = SCP[4 * p, 1]
            nw_b = SCP[4 * p, 2]
        else:
            s_b = T1[4 + p, 0]
            ne_b = T1[4 + p, 1]
            nw_b = T1[4 + p, 2]
        hd_f = jnp.where((nw_f > 0) & (ne_f > 0),
                         s_f / jnp.maximum(ca, 1.0), 0.0)
        hd_b = jnp.where((nw_b > 0) & (ne_b > 0),
                         s_b / jnp.maximum(cb, 1.0), 0.0)
        hh = jnp.maximum(hd_f, hd_b)
        hd[p] = jnp.where(ca > 0, hh, 32.0)
        fail[p] = jnp.where(ca > 0, 0.0, 1.0)

    f1 = fail[0] + fail[1] + fail[2] + fail[3]
    f2 = fail[4] + fail[5] + fail[6] + fail[7]

    rr = lax.broadcasted_iota(jnp.int32, (8, 128), 0)
    cc = lax.broadcasted_iota(jnp.int32, (8, 128), 1)
    hdpad = jnp.zeros((8, 128), jnp.float32)
    for i in range(4):
        h1 = hd[i]
        h2 = hd[4 + i]
        for col, val in [(1, h1), (2, h2), (3, (h1 + h2) / 3.0),
                         (4, h1 / 2.0)]:
            hdpad = hdpad + jnp.where((rr == i) & (cc == col), val, 0.0)
    hd_ref[...] = hdpad

    fpad = jnp.zeros((8, 128), jnp.float32)
    for col, val in [(1, f1), (2, f2), (3, (f1 + f2) / 3.0),
                     (4, (f1 + f2) / 2.0)]:
        fpad = fpad + jnp.where((rr == 0) & (cc == col), val, 0.0)
    fail_ref[...] = fpad


def kernel(predictions, labels):
    sc_partials = _sc_call(predictions.reshape(-1), labels.reshape(-1))
    t1 = pl.pallas_call(
        _tc_fwd_body,
        out_shape=jax.ShapeDtypeStruct((16, 128), jnp.float32),
    )(predictions, labels)
    hdpad, fpad = pl.pallas_call(
        _asm_body,
        out_shape=[
            jax.ShapeDtypeStruct((8, 128), jnp.float32),
            jax.ShapeDtypeStruct((8, 128), jnp.float32),
        ],
    )(t1, sc_partials.reshape(16, 16))
    return hdpad[:4, :5], fpad[0, :5]


# speedup vs baseline: 1.2406x; 1.0481x over previous
import jax
import jax.numpy as jnp
from jax import lax
from jax.experimental import pallas as pl
from jax.experimental.pallas import tpu as pltpu
from jax.experimental.pallas import tpu_sc as plsc

_BIG = 1e9
_SENT = 1.0e4
_INIT = 1.0e9


def _sc_sqrt(v):
    iv = plsc.bitcast(v, jnp.int32)
    r = plsc.bitcast(0x5F3759DF - (iv >> 1), jnp.float32)
    for _ in range(3):
        r = r * (1.5 - 0.5 * v * r * r)
    return v * r


def _sc_body(pred_hbm, lab_hbm, out_hbm, predv, labv, ffv, wfv, bnv, sbv,
             gtv, outv):
    io = lax.iota(jnp.int32, 16)
    m = lax.axis_index("s")
    i_img = m // 4
    q = m % 4

    pltpu.sync_copy(pred_hbm.at[pl.ds(i_img * 12288, 12288)], predv)
    pltpu.sync_copy(
        lab_hbm.at[pl.ds((i_img * 3 + 1) * 4096, 4096)], labv)

    z16 = jnp.zeros((16,), jnp.float32)
    s16 = jnp.full((16,), _SENT, jnp.float32)

    lms = [jnp.where(io + xg * 16 > 0, 1.0, 0.0) for xg in range(4)]
    rms = [jnp.where(io + xg * 16 < 63, 1.0, 0.0) for xg in range(4)]

    def _build_row(y):
        arow = []
        brow = []
        for xg in range(4):
            off = y * 64 + xg * 16
            p0 = predv[pl.ds(off, 16)]
            p1 = predv[pl.ds(4096 + off, 16)]
            p2 = predv[pl.ds(8192 + off, 16)]
            b = labv[pl.ds(off, 16)].astype(jnp.float32)
            a = jnp.where((p1 > p0) & (p1 >= p2), 1.0, 0.0)
            w = b * (1.0 - a)
            ffv[pl.ds(off, 16)] = a
            wfv[pl.ds(off, 16)] = w
            arow.append(a)
            brow.append(b)
        return arow, brow

    def _finish_row(yq, a_prev, a_cur, a_next, f, ne, um, dm):
        f_out = []
        for xg in range(4):
            off = yq * 64 + xg * 16
            c = a_cur[xg]
            up = a_prev[xg] * um
            dn = a_next[xg] * dm
            xi = off + io
            li = jnp.maximum(xi - 1, yq * 64)
            ri = jnp.minimum(xi + 1, yq * 64 + 63)
            lf = plsc.load_gather(ffv, [li]) * lms[xg]
            rf = plsc.load_gather(ffv, [ri]) * rms[xg]
            nb = c + up + dn + lf + rf
            bv = jnp.where(c * (5.0 - nb) > 0.0, 1.0, 0.0)
            bnv[pl.ds(off, 16)] = bv
            ne = ne + bv
            fn = (f[xg] + 1.0) * (1.0 - bv)
            sbv[pl.ds(off, 16)] = fn
            f_out.append(fn)
        return tuple(f_out), ne

    a0, b0 = _build_row(0)
    ca0 = a0[0] + a0[1] + a0[2] + a0[3]
    cb0 = b0[0] + b0[1] + b0[2] + b0[3]
    nw0 = (b0[0] * (1.0 - a0[0]) + b0[1] * (1.0 - a0[1])
           + b0[2] * (1.0 - a0[2]) + b0[3] * (1.0 - a0[3]))

    def fused_body(y, carry):
        a_prev, a_cur, f, ca, cb, nw, ne = carry
        um = jnp.where(jnp.full((16,), y, jnp.int32) > 1, 1.0, 0.0)
        a_next, brow = _build_row(y)
        for xg in range(4):
            ca = ca + a_next[xg]
            cb = cb + brow[xg]
            nw = nw + brow[xg] * (1.0 - a_next[xg])
        f, ne = _finish_row(y - 1, a_prev, a_cur, tuple(a_next), f, ne,
                            um, jnp.full((16,), 1.0))
        return (a_cur, tuple(a_next), f, ca, cb, nw, ne)

    zrow = (z16, z16, z16, z16)
    carry = (zrow, tuple(a0), (s16, s16, s16, s16), ca0, cb0, nw0, z16)
    a_prev, a_cur, f_v, ca_v, cb_v, nw_v, ne_v = lax.fori_loop(
        1, 64, fused_body, carry)
    _, ne_v = _finish_row(63, a_prev, a_cur, zrow, f_v, ne_v,
                          jnp.full((16,), 1.0), jnp.full((16,), 0.0))

    def p4_body(t, bw):
        y = 63 - t
        out = []
        for xg in range(4):
            off = y * 64 + xg * 16
            pen = bnv[pl.ds(off, 16)]
            bn = (bw[xg] + 1.0) * (1.0 - pen)
            near = jnp.minimum(bn, sbv[pl.ds(off, 16)])
            g = near * near
            idx = io * 64 + (xg * 1024 + y)
            plsc.store_scatter(gtv, [idx], g)
            out.append(bn)
        return tuple(out)

    lax.fori_loop(0, 64, p4_body, (s16, s16, s16, s16))

    x0 = q * 16

    def chunk_body(cidx, acc):
        xb = x0 + cidx * 8
        init = tuple(jnp.full((16,), _INIT, jnp.float32) for _ in range(32))

        def inner(xp, st):
            rows = [gtv[pl.ds(xp * 64 + q * 16, 16)] for q in range(4)]
            base = (xb - xp).astype(jnp.float32)
            new = []
            for k in range(8):
                dk = base + float(k)
                add = jnp.full((16,), dk * dk)
                for q in range(4):
                    new.append(jnp.minimum(st[k * 4 + q], rows[q] + add))
            return tuple(new)

        st = lax.fori_loop(0, 64, inner, init)
        for k in range(8):
            x = xb + k
            for q in range(4):
                s = _sc_sqrt(st[k * 4 + q])
                widx = io * 64 + (q * 1024 + x)
                wv = plsc.load_gather(wfv, [widx])
                acc = acc + s * wv
        return acc

    acc_v = lax.fori_loop(0, 2, chunk_body, z16)

    res = jnp.where(io == 0, jnp.sum(acc_v), 0.0)
    res = res + jnp.where(io == 1, jnp.sum(ne_v), 0.0)
    res = res + jnp.where(io == 2, jnp.sum(nw_v), 0.0)
    res = res + jnp.where(io == 3, jnp.sum(ca_v), 0.0)
    res = res + jnp.where(io == 4, jnp.sum(cb_v), 0.0)
    outv[...] = res
    pltpu.sync_copy(outv, out_hbm.at[pl.ds(m * 16, 16)])


_sc_call = pl.kernel(
    _sc_body,
    out_type=jax.ShapeDtypeStruct((256,), jnp.float32),
    mesh=plsc.VectorSubcoreMesh(core_axis_name="c", subcore_axis_name="s",
                                num_cores=1, num_subcores=16),
    compiler_params=pltpu.CompilerParams(needs_layout_passes=False),
    scratch_types=[
        pltpu.VMEM((12288,), jnp.float32),
        pltpu.VMEM((4096,), jnp.int32),
        pltpu.VMEM((4096,), jnp.float32),
        pltpu.VMEM((4096,), jnp.float32),
        pltpu.VMEM((4096,), jnp.float32),
        pltpu.VMEM((4096,), jnp.float32),
        pltpu.VMEM((4096,), jnp.float32),
        pltpu.VMEM((16,), jnp.float32),
    ],
)


def _boundary_mask(mk):
    z_row = jnp.zeros((1, 64), jnp.float32)
    z_col = jnp.zeros((64, 1), jnp.float32)
    new = mk
    new = new + jnp.concatenate([mk[1:, :], z_row], axis=0)
    new = new + jnp.concatenate([z_row, mk[:-1, :]], axis=0)
    new = new + jnp.concatenate([mk[:, 1:], z_col], axis=1)
    new = new + jnp.concatenate([z_col, mk[:, :-1]], axis=1)
    return jnp.where(mk * (5.0 - new) > 0.0, 1.0, 0.0)


def _minplus_pass(pen):
    t_idx = jax.lax.broadcasted_iota(jnp.int32, (64, 1), 0).astype(jnp.float32)
    out = jnp.full(pen.shape, 4.0 * _BIG, jnp.float32)
    for s in range(64):
        d2 = (t_idx - float(s)) ** 2
        out = jnp.minimum(out, pen[s:s + 1, :] + d2)
    return out


def _transpose_blocks(x, eye, nblk):
    outs = []
    for mi in range(nblk):
        blk = x[:, mi * 64:(mi + 1) * 64]
        outs.append(
            jax.lax.dot_general(blk, eye, (((0,), (0,)), ((), ())),
                                preferred_element_type=jnp.float32))
    return jnp.concatenate(outs, axis=1)


def _tc_fwd_body(pred_ref, lab_ref, out_ref):
    pred = pred_ref[...]
    lab = lab_ref[...].astype(jnp.float32)

    p0, p1, p2 = pred[:, 0], pred[:, 1], pred[:, 2]
    a_cls = [
        None,
        jnp.where((p1 > p0) & (p1 >= p2), 1.0, 0.0),
        jnp.where((p2 > p0) & (p2 > p1), 1.0, 0.0),
    ]

    edt_masks = [None] * 12
    w_masks = [None] * 12
    count_a = [None] * 8
    count_b = [None] * 8
    for jidx in range(2):
        for i in range(4):
            p = jidx * 4 + i
            a = a_cls[jidx + 1][i]
            b = lab[i, jidx + 1]
            edt_masks[p] = _boundary_mask(b)
            w_masks[p] = a * (1.0 - b)
            count_a[p] = jnp.sum(a)
            count_b[p] = jnp.sum(b)
            if jidx == 1:
                edt_masks[8 + i] = _boundary_mask(a)
                w_masks[8 + i] = b * (1.0 - a)

    pen = jnp.concatenate([(1.0 - mm) * _BIG for mm in edt_masks], axis=1)
    w2d = jnp.concatenate(w_masks, axis=1)

    r64 = jax.lax.broadcasted_iota(jnp.int32, (64, 64), 0)
    c64 = jax.lax.broadcasted_iota(jnp.int32, (64, 64), 1)
    eye = (r64 == c64).astype(jnp.float32)

    g = _minplus_pass(pen)
    gt = _transpose_blocks(g, eye, 12)
    d2t = _minplus_pass(gt)
    wt = _transpose_blocks(w2d, eye, 12)

    prod = jnp.sqrt(jnp.maximum(d2t, 0.0)) * wt

    rr = lax.broadcasted_iota(jnp.int32, (16, 128), 0)
    cc = lax.broadcasted_iota(jnp.int32, (16, 128), 1)
    out = jnp.zeros((16, 128), jnp.float32)
    for p in range(12):
        blk = slice(p * 64, (p + 1) * 64)
        s_p = jnp.sum(prod[:, blk])
        ne_p = jnp.sum(pen[:, blk] < 1.0)
        nw_p = jnp.sum(w2d[:, blk])
        vals = [(0, s_p), (1, ne_p), (2, nw_p)]
        if p < 8:
            vals += [(3, count_a[p]), (4, count_b[p])]
        for col, val in vals:
            out = out + jnp.where((rr == p) & (cc == col), val, 0.0)
    out_ref[...] = out


def _asm_body(t1_ref, scp_ref, hd_ref, fail_ref):
    T1 = t1_ref[...]
    SCP = scp_ref[...]
    hd = [None] * 8
    fail = [None] * 8
    for p in range(8):
        s_f = T1[p, 0]
        ne_f = T1[p, 1]
        nw_f = T1[p, 2]
        ca = T1[p, 3]
        cb = T1[p, 4]
        if p < 4:
            s_b = (SCP[4 * p, 0] + SCP[4 * p + 1, 0] + SCP[4 * p + 2, 0]
                   + SCP[4 * p + 3, 0])
            ne_b = SCP[4 * p, 1]
            nw_b = SCP[4 * p, 2]
        else:
            s_b = T1[4 + p, 0]
            ne_b = T1[4 + p, 1]
            nw_b = T1[4 + p, 2]
        hd_f = jnp.where((nw_f > 0) & (ne_f > 0),
                         s_f / jnp.maximum(ca, 1.0), 0.0)
        hd_b = jnp.where((nw_b > 0) & (ne_b > 0),
                         s_b / jnp.maximum(cb, 1.0), 0.0)
        hh = jnp.maximum(hd_f, hd_b)
        hd[p] = jnp.where(ca > 0, hh, 32.0)
        fail[p] = jnp.where(ca > 0, 0.0, 1.0)

    f1 = fail[0] + fail[1] + fail[2] + fail[3]
    f2 = fail[4] + fail[5] + fail[6] + fail[7]

    rr = lax.broadcasted_iota(jnp.int32, (8, 128), 0)
    cc = lax.broadcasted_iota(jnp.int32, (8, 128), 1)
    hdpad = jnp.zeros((8, 128), jnp.float32)
    for i in range(4):
        h1 = hd[i]
        h2 = hd[4 + i]
        for col, val in [(1, h1), (2, h2), (3, (h1 + h2) / 3.0),
                         (4, h1 / 2.0)]:
            hdpad = hdpad + jnp.where((rr == i) & (cc == col), val, 0.0)
    hd_ref[...] = hdpad

    fpad = jnp.zeros((8, 128), jnp.float32)
    for col, val in [(1, f1), (2, f2), (3, (f1 + f2) / 3.0),
                     (4, (f1 + f2) / 2.0)]:
        fpad = fpad + jnp.where((rr == 0) & (cc == col), val, 0.0)
    fail_ref[...] = fpad


def kernel(predictions, labels):
    sc_partials = _sc_call(predictions.reshape(-1), labels.reshape(-1))
    t1 = pl.pallas_call(
        _tc_fwd_body,
        out_shape=jax.ShapeDtypeStruct((16, 128), jnp.float32),
    )(predictions, labels)
    hdpad, fpad = pl.pallas_call(
        _asm_body,
        out_shape=[
            jax.ShapeDtypeStruct((8, 128), jnp.float32),
            jax.ShapeDtypeStruct((8, 128), jnp.float32),
        ],
    )(t1, sc_partials.reshape(16, 16))
    return hdpad[:4, :5], fpad[0, :5]
